# Initial kernel scaffold; baseline (speedup 1.0000x reference)
import jax
import jax.numpy as jnp
from jax.experimental import pallas as pl


def kernel(Pb, V, bias):
    raise NotImplementedError("write your pallas kernel here")


# speedup vs baseline: 16.0113x; 16.0113x over previous
import functools
import jax
import jax.numpy as jnp
from jax.experimental import pallas as pl
from jax.experimental.pallas import tpu as pltpu

_T = 1024
_E = 64
_F = 768


def _body(p_ref, v_ref, b_ref, o_ref):
    p = p_ref[...]
    s = p + b_ref[...]
    ii = jax.lax.broadcasted_iota(jnp.int32, p.shape, 1)

    m1 = jnp.max(s, axis=1, keepdims=True)
    i1 = jnp.min(jnp.where(s == m1, ii, _E), axis=1, keepdims=True)
    sel1 = ii == i1
    l1 = jnp.max(jnp.where(sel1, p, -jnp.inf), axis=1, keepdims=True)

    s2 = jnp.where(sel1, -jnp.inf, s)
    m2 = jnp.max(s2, axis=1, keepdims=True)
    i2 = jnp.min(jnp.where(s2 == m2, ii, _E), axis=1, keepdims=True)
    sel2 = ii == i2
    l2 = jnp.max(jnp.where(sel2, p, -jnp.inf), axis=1, keepdims=True)

    m = jnp.maximum(l1, l2)
    e1 = jnp.exp(l1 - m)
    e2 = jnp.exp(l2 - m)
    inv = 1.0 / (e1 + e2)
    w1 = e1 * inv
    w2 = e2 * inv

    w = jnp.where(sel1, w1, 0.0) + jnp.where(sel2, w2, 0.0)
    o_ref[...] = jnp.dot(w, v_ref[...], preferred_element_type=jnp.float32)


@functools.partial(jax.jit, static_argnames=())
def _run(p2d, V, b2d):
    n = p2d.shape[0]
    grid = (n // _T,)
    return pl.pallas_call(
        _body,
        grid=grid,
        in_specs=[
            pl.BlockSpec((_T, _E), lambda i: (i, 0)),
            pl.BlockSpec((_E, _F), lambda i: (0, 0)),
            pl.BlockSpec((1, _E), lambda i: (0, 0)),
        ],
        out_specs=pl.BlockSpec((_T, _F), lambda i: (i, 0)),
        out_shape=jax.ShapeDtypeStruct((n, _F), jnp.float32),
    )(p2d, V, b2d)


def kernel(Pb, V, bias):
    B, r, E = Pb.shape
    p2d = Pb.astype(jnp.float32).reshape(B * r, E)
    out = _run(p2d, V, bias.reshape(1, E))
    return out.reshape(B, r, V.shape[1]).astype(V.dtype)
